# baseline (speedup 1.0000x reference)
import jax
import jax.numpy as jnp
from jax import lax
from jax.experimental import pallas as pl
from jax.experimental.pallas import tpu as pltpu
from jax.experimental.pallas import tpu_sc as plsc

N = 10000
E = 320000
D = 128
NC = 2
NS = 16
NW = NC * NS
EPW = E // NW
B = 80
NCH = 125
NVR = D // 16
RPT = 624
ZR = 104


def _feats_body(x_ref, pos_ref, wl_ref, wr_ref, we_ref, att_ref,
                u_ref, v_ref, xl_ref):
    x = x_ref[...]
    p = pos_ref[...] @ we_ref[...]
    xl = x @ wl_ref[...]
    xr = x @ wr_ref[...]
    a6 = 0.6 * att_ref[...]
    u_ref[...] = (xl - p) * a6
    v_ref[...] = (xr + p) * a6
    xl_ref[...] = xl


def _feats(x, pos, W_l, W_r, W_e, att):
    bn = 2000
    grid = N // bn
    return pl.pallas_call(
        _feats_body,
        grid=(grid,),
        in_specs=[
            pl.BlockSpec((bn, D), lambda i: (i, 0)),
            pl.BlockSpec((bn, 3), lambda i: (i, 0)),
            pl.BlockSpec((D, D), lambda i: (0, 0)),
            pl.BlockSpec((D, D), lambda i: (0, 0)),
            pl.BlockSpec((3, D), lambda i: (0, 0)),
            pl.BlockSpec((1, D), lambda i: (0, 0)),
        ],
        out_specs=[
            pl.BlockSpec((bn, D), lambda i: (i, 0)),
            pl.BlockSpec((bn, D), lambda i: (i, 0)),
            pl.BlockSpec((bn, D), lambda i: (i, 0)),
        ],
        out_shape=[
            jax.ShapeDtypeStruct((N, D), jnp.float32),
            jax.ShapeDtypeStruct((N, D), jnp.float32),
            jax.ShapeDtypeStruct((N, D), jnp.float32),
        ],
    )(x, pos, W_l, W_r, W_e, att)


def _edges_body(u_h, v_h, xl_h, att_h, sdm_h,
                outp_h, denp_h,
                idx_v, att_v, urows, vrows, xlrows,
                red_v, ex_v, zrows, zvec, out_sh, den_sh,
                sem_g, sem_x, sem_s, sem_i):
    c = lax.axis_index("c")
    s = lax.axis_index("s")
    wid = c * NS + s

    pltpu.sync_copy(att_h, att_v)

    zero16 = jnp.zeros((16,), jnp.float32)

    def _zrow_body(i, _):
        r = i // NVR
        j = i % NVR
        zrows[r, pl.ds(j * 16, 16)] = zero16
        return 0

    lax.fori_loop(0, ZR * NVR, _zrow_body, 0)

    def _zvec_body(i, _):
        zvec[pl.ds(i * 16, 16)] = zero16
        return 0

    lax.fori_loop(0, 1000 // 16, _zvec_body, 0)

    for i in range(RPT // ZR):
        pltpu.sync_copy(zrows, out_sh.at[pl.ds(s * RPT + i * ZR, ZR)])

    @pl.when(s == 0)
    def _():
        pltpu.sync_copy(zrows.at[pl.ds(0, N - NS * RPT)],
                        out_sh.at[pl.ds(NS * RPT, N - NS * RPT)])
        for i in range(N // 1000):
            pltpu.sync_copy(zvec, den_sh.at[pl.ds(i * 1000, 1000)])

    plsc.subcore_barrier()

    cpos = jnp.full((16,), 2.0 / 3.0, jnp.float32)
    sg_regs = [jnp.where(att_v[pl.ds(j * 16, 16)] > 0, cpos, -cpos)
               for j in range(NVR)]
    lane0 = lax.iota(jnp.int32, 16)
    zero16i = jnp.zeros((16,), jnp.int32)

    def lane_sum(acc):
        for kk in (8, 4, 2, 1):
            idx = (lane0 + kk) & 15
            acc = acc + jnp.take_along_axis(acc, idx, axis=0,
                                            mode="promise_in_bounds")
        return acc

    def idx_load(k):
        pltpu.async_copy(sdm_h.at[wid, k], idx_v.at[k & 1], sem_i)

    def idx_drain():
        pltpu.make_async_copy(sdm_h.at[0, 0], idx_v.at[0], sem_i).wait()

    def edge_loop():
        def edge_body(i, _):
            e0 = i * 2
            e1 = e0 + 1
            acc0a = acc0b = acc1a = acc1b = zero16
            for j in range(NVR):
                sl = pl.ds(j * 16, 16)
                t0 = urows[e0, sl] + vrows[e0, sl]
                t1 = urows[e1, sl] + vrows[e1, sl]
                w0 = t0 + sg_regs[j] * jnp.abs(t0)
                w1 = t1 + sg_regs[j] * jnp.abs(t1)
                if j % 2 == 0:
                    acc0a = acc0a + w0
                    acc1a = acc1a + w1
                else:
                    acc0b = acc0b + w0
                    acc1b = acc1b + w1
            red_v[pl.ds(e0 * 16, 16)] = lane_sum(acc0a + acc0b)
            red_v[pl.ds(e1 * 16, 16)] = lane_sum(acc1a + acc1b)
            return 0

        lax.fori_loop(0, B // 2, edge_body, 0)

    def group_loop(k):
        q = k & 1

        def grp_body(g, _):
            e16 = g * 16 + lane0
            ex16 = jnp.exp(plsc.load_gather(red_v, [e16 * 16]))
            ex_v[q, pl.ds(g * 16, 16)] = ex16
            return 0

        lax.fori_loop(0, B // 16, grp_body, 0)

    def scale_loop(k):
        q = k & 1

        def sc_body(g, _):
            ex16 = ex_v[q, pl.ds(g * 16, 16)]
            for l in range(16):
                e = g * 16 + l
                sc = ex16[l]
                for j in range(NVR):
                    sl = pl.ds(j * 16, 16)
                    xlrows[e, sl] = xlrows[e, sl] * sc
            return 0

        lax.fori_loop(0, B // 16, sc_body, 0)

    def chunk_work(k):
        q = k & 1
        cpu_ = pltpu.async_copy(u_h.at[idx_v.at[q, 0]], urows, sem_g)
        cpv_ = pltpu.async_copy(v_h.at[idx_v.at[q, 1]], vrows, sem_g)
        cpx_ = pltpu.async_copy(xl_h.at[idx_v.at[q, 0]], xlrows, sem_x)
        cpu_.wait()
        cpv_.wait()
        edge_loop()
        group_loop(k)
        cpx_.wait()
        scale_loop(k)
        cd = pltpu.async_copy(ex_v.at[q], den_sh.at[idx_v.at[q, 1]], sem_s,
                              add=True)
        co = pltpu.async_copy(xlrows, out_sh.at[idx_v.at[q, 1]], sem_s,
                              add=True)
        cd.wait()
        co.wait()

    idx_load(0)

    def chunk_body(k, _):
        idx_drain()
        idx_load(k + 1)
        chunk_work(k)
        return 0

    lax.fori_loop(0, NCH - 1, chunk_body, 0)

    idx_drain()
    chunk_work(NCH - 1)

    plsc.subcore_barrier()

    pltpu.sync_copy(out_sh.at[pl.ds(s * RPT, RPT)],
                    outp_h.at[c, pl.ds(s * RPT, RPT)])

    @pl.when(s == 0)
    def _():
        pltpu.sync_copy(den_sh, denp_h.at[c])
        pltpu.sync_copy(out_sh.at[pl.ds(NS * RPT, N - NS * RPT)],
                        outp_h.at[c, pl.ds(NS * RPT, N - NS * RPT)])


def _edges(u, v, xl, att1d, sdm):
    mesh = plsc.VectorSubcoreMesh(core_axis_name="c", subcore_axis_name="s")
    f = pl.kernel(
        _edges_body,
        out_type=[
            jax.ShapeDtypeStruct((NC, N, D), jnp.float32),
            jax.ShapeDtypeStruct((NC, N), jnp.float32),
        ],
        mesh=mesh,
        compiler_params=pltpu.CompilerParams(needs_layout_passes=False),
        scratch_types=[
            pltpu.VMEM((2, 2, B), jnp.int32),
            pltpu.VMEM((D,), jnp.float32),
            pltpu.VMEM((B, D), jnp.float32),
            pltpu.VMEM((B, D), jnp.float32),
            pltpu.VMEM((B, D), jnp.float32),
            pltpu.VMEM((B * 16,), jnp.float32),
            pltpu.VMEM((2, B), jnp.float32),
            pltpu.VMEM((ZR, D), jnp.float32),
            pltpu.VMEM((1000,), jnp.float32),
            pltpu.VMEM_SHARED((N, D), jnp.float32),
            pltpu.VMEM_SHARED((N,), jnp.float32),
            pltpu.SemaphoreType.DMA,
            pltpu.SemaphoreType.DMA,
            pltpu.SemaphoreType.DMA,
            pltpu.SemaphoreType.DMA,
        ],
    )
    return f(u, v, xl, att1d, sdm)


def _finalize_body(outp_ref, denp_ref, gamma_ref, beta_ref, o_ref):
    p = outp_ref[0] + outp_ref[1]
    dsum = (denp_ref[0] + denp_ref[1] + 1e-16).reshape(N, 1)
    out = p / dsum
    mean = jnp.mean(out, axis=0, keepdims=True)
    var = jnp.mean((out - mean) ** 2, axis=0, keepdims=True)
    o_ref[...] = (out - mean) / jnp.sqrt(var + 1e-5) * gamma_ref[...] + beta_ref[...]


def _finalize(outp, denp, gamma, beta):
    return pl.pallas_call(
        _finalize_body,
        out_shape=jax.ShapeDtypeStruct((N, D), jnp.float32),
    )(outp, denp.reshape(NC, N, 1), gamma.reshape(1, D), beta.reshape(1, D))


def kernel(x, pos, edge_index, W_l, W_r, W_e, att, gamma, beta):
    sdm = jnp.stack([edge_index[0].reshape(NW, NCH, B),
                     edge_index[1].reshape(NW, NCH, B)],
                    axis=2)
    u, v, xl = _feats(x, pos, W_l, W_r, W_e, att.reshape(1, D))
    outp, denp = _edges(u, v, xl, att.reshape(D), sdm)
    return _finalize(outp, denp, gamma, beta)


# speedup vs baseline: 1.1616x; 1.1616x over previous
import jax
import jax.numpy as jnp
from jax import lax
from jax.experimental import pallas as pl
from jax.experimental.pallas import tpu as pltpu
from jax.experimental.pallas import tpu_sc as plsc

N = 10000
E = 320000
D = 128
NC = 2
NS = 16
NW = NC * NS
EPW = E // NW
B = 80
NCH = 125
NVR = D // 16
RPT = 624
ZR = 8


def _feats_body(x_ref, pos_ref, wl_ref, wr_ref, we_ref, att_ref,
                u_ref, v_ref, xl_ref):
    x = x_ref[...]
    p = pos_ref[...] @ we_ref[...]
    xl = x @ wl_ref[...]
    xr = x @ wr_ref[...]
    a6 = 0.6 * att_ref[...]
    u_ref[...] = (xl - p) * a6
    v_ref[...] = (xr + p) * a6
    xl_ref[...] = xl


def _feats(x, pos, W_l, W_r, W_e, att):
    bn = 2000
    grid = N // bn
    return pl.pallas_call(
        _feats_body,
        grid=(grid,),
        in_specs=[
            pl.BlockSpec((bn, D), lambda i: (i, 0)),
            pl.BlockSpec((bn, 3), lambda i: (i, 0)),
            pl.BlockSpec((D, D), lambda i: (0, 0)),
            pl.BlockSpec((D, D), lambda i: (0, 0)),
            pl.BlockSpec((3, D), lambda i: (0, 0)),
            pl.BlockSpec((1, D), lambda i: (0, 0)),
        ],
        out_specs=[
            pl.BlockSpec((bn, D), lambda i: (i, 0)),
            pl.BlockSpec((bn, D), lambda i: (i, 0)),
            pl.BlockSpec((bn, D), lambda i: (i, 0)),
        ],
        out_shape=[
            jax.ShapeDtypeStruct((N, D), jnp.float32),
            jax.ShapeDtypeStruct((N, D), jnp.float32),
            jax.ShapeDtypeStruct((N, D), jnp.float32),
        ],
    )(x, pos, W_l, W_r, W_e, att)


def _edges_body(u_h, v_h, xl_h, att_h, sdm_h,
                outp_h, denp_h,
                idx_v, att_v, urows, vrows, xlrows,
                red_v, ex_v, zrows, zvec, out_sh, den_sh,
                sem_g, sem_x, sem_s, sem_i):
    c = lax.axis_index("c")
    s = lax.axis_index("s")
    wid = c * NS + s

    pltpu.sync_copy(att_h, att_v)

    zero16 = jnp.zeros((16,), jnp.float32)

    def _zrow_body(i, _):
        r = i // NVR
        j = i % NVR
        zrows[r, pl.ds(j * 16, 16)] = zero16
        return 0

    lax.fori_loop(0, ZR * NVR, _zrow_body, 0)

    def _zvec_body(i, _):
        zvec[pl.ds(i * 16, 16)] = zero16
        return 0

    lax.fori_loop(0, 1000 // 16, _zvec_body, 0)

    for i in range(RPT // ZR):
        pltpu.sync_copy(zrows, out_sh.at[pl.ds(s * RPT + i * ZR, ZR)])

    @pl.when(s == 0)
    def _():
        for i in range((N - NS * RPT) // ZR):
            pltpu.sync_copy(zrows, out_sh.at[pl.ds(NS * RPT + i * ZR, ZR)])
        for i in range(N // 1000):
            pltpu.sync_copy(zvec, den_sh.at[pl.ds(i * 1000, 1000)])

    plsc.subcore_barrier()

    cpos = jnp.full((16,), 2.0 / 3.0, jnp.float32)
    sg_regs = [jnp.where(att_v[pl.ds(j * 16, 16)] > 0, cpos, -cpos)
               for j in range(NVR)]
    lane0 = lax.iota(jnp.int32, 16)
    zero16i = jnp.zeros((16,), jnp.int32)

    def lane_sum(acc):
        for kk in (8, 4, 2, 1):
            idx = (lane0 + kk) & 15
            acc = acc + jnp.take_along_axis(acc, idx, axis=0,
                                            mode="promise_in_bounds")
        return acc

    def idx_load(k):
        pltpu.async_copy(sdm_h.at[wid, k], idx_v.at[k & 1], sem_i)

    def idx_drain():
        pltpu.make_async_copy(sdm_h.at[0, 0], idx_v.at[0], sem_i).wait()

    def edge_loop():
        def edge_body(i, _):
            e0 = i * 2
            e1 = e0 + 1
            acc0a = acc0b = acc1a = acc1b = zero16
            for j in range(NVR):
                sl = pl.ds(j * 16, 16)
                t0 = urows[e0, sl] + vrows[e0, sl]
                t1 = urows[e1, sl] + vrows[e1, sl]
                w0 = t0 + sg_regs[j] * jnp.abs(t0)
                w1 = t1 + sg_regs[j] * jnp.abs(t1)
                if j % 2 == 0:
                    acc0a = acc0a + w0
                    acc1a = acc1a + w1
                else:
                    acc0b = acc0b + w0
                    acc1b = acc1b + w1
            red_v[e0, :] = lane_sum(acc0a + acc0b)
            red_v[e1, :] = lane_sum(acc1a + acc1b)
            return 0

        lax.fori_loop(0, B // 2, edge_body, 0)

    def group_loop(k):
        q = k & 1

        def grp_body(g, _):
            e16 = g * 16 + lane0
            ex16 = jnp.exp(plsc.load_gather(red_v, [e16, zero16i]))
            ex_v[q, pl.ds(g * 16, 16)] = ex16
            return 0

        lax.fori_loop(0, B // 16, grp_body, 0)

    def scale_loop(k):
        q = k & 1

        def sc_body(g, _):
            ex16 = ex_v[q, pl.ds(g * 16, 16)]
            for l in range(16):
                e = g * 16 + l
                sc = ex16[l]
                for j in range(NVR):
                    sl = pl.ds(j * 16, 16)
                    xlrows[e, sl] = xlrows[e, sl] * sc
            return 0

        lax.fori_loop(0, B // 16, sc_body, 0)

    def chunk_work(k):
        q = k & 1
        cpu_ = pltpu.async_copy(u_h.at[idx_v.at[q, 0]], urows, sem_g)
        cpv_ = pltpu.async_copy(v_h.at[idx_v.at[q, 1]], vrows, sem_g)
        cpx_ = pltpu.async_copy(xl_h.at[idx_v.at[q, 0]], xlrows, sem_x)
        cpu_.wait()
        cpv_.wait()
        edge_loop()
        group_loop(k)
        cpx_.wait()
        scale_loop(k)
        cd = pltpu.async_copy(ex_v.at[q], den_sh.at[idx_v.at[q, 1]], sem_s,
                              add=True)
        co = pltpu.async_copy(xlrows, out_sh.at[idx_v.at[q, 1]], sem_s,
                              add=True)
        cd.wait()
        co.wait()

    idx_load(0)

    def chunk_body(k, _):
        idx_drain()
        idx_load(k + 1)
        chunk_work(k)
        return 0

    lax.fori_loop(0, NCH - 1, chunk_body, 0)

    idx_drain()
    chunk_work(NCH - 1)

    plsc.subcore_barrier()

    pltpu.sync_copy(out_sh.at[pl.ds(s * RPT, RPT)],
                    outp_h.at[c, pl.ds(s * RPT, RPT)])

    @pl.when(s == 0)
    def _():
        pltpu.sync_copy(den_sh, denp_h.at[c])
        pltpu.sync_copy(out_sh.at[pl.ds(NS * RPT, N - NS * RPT)],
                        outp_h.at[c, pl.ds(NS * RPT, N - NS * RPT)])


def _edges(u, v, xl, att1d, sdm):
    mesh = plsc.VectorSubcoreMesh(core_axis_name="c", subcore_axis_name="s")
    f = pl.kernel(
        _edges_body,
        out_type=[
            jax.ShapeDtypeStruct((NC, N, D), jnp.float32),
            jax.ShapeDtypeStruct((NC, N), jnp.float32),
        ],
        mesh=mesh,
        compiler_params=pltpu.CompilerParams(needs_layout_passes=False),
        scratch_types=[
            pltpu.VMEM((2, 2, B), jnp.int32),
            pltpu.VMEM((D,), jnp.float32),
            pltpu.VMEM((B, D), jnp.float32),
            pltpu.VMEM((B, D), jnp.float32),
            pltpu.VMEM((B, D), jnp.float32),
            pltpu.VMEM((B, 16), jnp.float32),
            pltpu.VMEM((2, B), jnp.float32),
            pltpu.VMEM((ZR, D), jnp.float32),
            pltpu.VMEM((1000,), jnp.float32),
            pltpu.VMEM_SHARED((N, D), jnp.float32),
            pltpu.VMEM_SHARED((N,), jnp.float32),
            pltpu.SemaphoreType.DMA,
            pltpu.SemaphoreType.DMA,
            pltpu.SemaphoreType.DMA,
            pltpu.SemaphoreType.DMA,
        ],
    )
    return f(u, v, xl, att1d, sdm)


def _finalize_body(outp_ref, denp_ref, gamma_ref, beta_ref, o_ref):
    p = outp_ref[0] + outp_ref[1]
    dsum = (denp_ref[0] + denp_ref[1] + 1e-16).reshape(N, 1)
    out = p / dsum
    mean = jnp.mean(out, axis=0, keepdims=True)
    var = jnp.mean((out - mean) ** 2, axis=0, keepdims=True)
    o_ref[...] = (out - mean) / jnp.sqrt(var + 1e-5) * gamma_ref[...] + beta_ref[...]


def _finalize(outp, denp, gamma, beta):
    return pl.pallas_call(
        _finalize_body,
        out_shape=jax.ShapeDtypeStruct((N, D), jnp.float32),
    )(outp, denp.reshape(NC, N, 1), gamma.reshape(1, D), beta.reshape(1, D))


def kernel(x, pos, edge_index, W_l, W_r, W_e, att, gamma, beta):
    sdm = jnp.stack([edge_index[0].reshape(NW, NCH, B),
                     edge_index[1].reshape(NW, NCH, B)],
                    axis=2)
    u, v, xl = _feats(x, pos, W_l, W_r, W_e, att.reshape(1, D))
    outp, denp = _edges(u, v, xl, att.reshape(D), sdm)
    return _finalize(outp, denp, gamma, beta)
